# baseline (speedup 1.0000x reference)
import jax
import jax.numpy as jnp
from jax.experimental import pallas as pl
from jax.experimental.pallas import tpu as pltpu

_BT = 1024
_ES = 64


def _leaky(x):
    return jnp.maximum(x, 0.01 * x)


def _dot(a, b):
    return jax.lax.dot(a, b, preferred_element_type=jnp.float32)


def _prep(typeE_ref, tableE_ref, colE_ref, opE_ref, joinE_ref,
          Wf_ref, Wh_ref, Wp_ref, T64_ref, W48_ref, Whh_ref):
    f32 = jnp.float32
    Wp = Wp_ref[...]
    Tt = _dot(typeE_ref[...], Wp[0:64, :])
    Tj = _dot(joinE_ref[...], Wp[137:201, :])
    Ttb = _dot(tableE_ref[...], Wp[201:265, :])
    T64_ref[...] = jnp.concatenate(
        [Tt, Tj, Ttb, jnp.zeros((14, 64), f32)], axis=0)

    Wf = Wf_ref[...]
    ct = _dot(colE_ref[...], Wf[0:64, :])
    ot = _dot(opE_ref[...], Wf[64:72, :])
    W48_ref[...] = jnp.concatenate(
        [ct, ot, Wf[72:73, :], jnp.zeros((7, 73), f32)], axis=0)

    Wh = Wh_ref[...]
    r150 = jax.lax.broadcasted_iota(jnp.int32, (150, 1), 0)
    c50 = jax.lax.broadcasted_iota(jnp.int32, (1, 50), 1)
    rep = _dot((r150 // 3 == c50).astype(f32), Wh)
    mod3 = r150 % 3
    strided = jnp.concatenate([jnp.where(mod3 == 0, rep, 0.0),
                               jnp.where(mod3 == 1, rep, 0.0),
                               jnp.where(mod3 == 2, rep, 0.0)], axis=1)
    Whh_ref[...] = jnp.concatenate(
        [jnp.zeros((14, 192), f32), strided, jnp.zeros((92, 192), f32)], axis=0)


def _block(x_ref, T64_ref, W48_ref, Wf2_ref, bf_ref, bf2_ref, Whh_ref, bh_ref,
           Ws_ref, bs_ref, Wpf_ref, Wptb_ref, Wph_ref, bp_ref, out_ref):
    f32 = jnp.float32
    i32 = jnp.int32

    ids = x_ref[:, 0:16]
    l64 = jax.lax.broadcasted_iota(i32, (1, 64), 1).astype(f32)
    oh64 = ((l64 == ids[:, 0:1]).astype(f32)
            + (l64 == ids[:, 1:2] + 20.0).astype(f32)
            + (l64 == x_ref[:, 164:165] + 30.0).astype(f32))
    acc = _dot(oh64, T64_ref[...])

    l48 = jax.lax.broadcasted_iota(i32, (1, 48), 1).astype(f32)
    W48 = W48_ref[...]
    Wf2 = Wf2_ref[...]
    bf = bf_ref[...]
    bf2 = bf2_ref[...]
    m = []
    ohs = []
    for j in range(3):
        m.append(ids[:, 11 + j:12 + j])
        ohs.append((l48 == ids[:, 2 + j:3 + j]).astype(f32)
                   + (l48 == ids[:, 5 + j:6 + j] + 30.0).astype(f32)
                   + ids[:, 8 + j:9 + j] * (l48 == 40.0).astype(f32))
    oh3 = jnp.concatenate(ohs, axis=0)
    h1 = _leaky(_dot(oh3, W48) + bf)
    h2 = _leaky(_dot(h1, Wf2) + bf2)
    h2m = jnp.concatenate(m, axis=0) * h2
    facc = h2m[0:_BT] + h2m[_BT:2 * _BT] + h2m[2 * _BT:3 * _BT]
    msum = m[0] + m[1] + m[2]
    rnum = 1.0 / jnp.maximum(msum, 1.0)

    hist3 = _dot(x_ref[:, 0:256], Whh_ref[...])
    histEmb = (m[0] * hist3[:, 0:64] + m[1] * hist3[:, 64:128]
               + m[2] * hist3[:, 128:192] + msum * bh_ref[...]) * rnum

    samp = _dot(x_ref[:, 128:1165], Ws_ref[...][0:1037, :])

    pre = (acc
           + _dot(facc * rnum, Wpf_ref[...])
           + _dot(samp + bs_ref[...], Wptb_ref[...])
           + _dot(histEmb, Wph_ref[...])
           + bp_ref[...])
    out_ref[...] = _leaky(pre)


def _full(w):
    return pl.BlockSpec(w.shape, lambda i: tuple(0 for _ in w.shape))


def kernel(feature, typeEmbed, tableEmbed, columnEmbed, opEmbed, joinEmbed,
           Wf, bf, Wf2, bf2, Ws, bs, Wh, bh, Wp, bp):
    B = feature.shape[0]
    f32 = jnp.float32

    T64, W48, Whh = pl.pallas_call(
        _prep,
        out_shape=(jax.ShapeDtypeStruct((64, 64), f32),
                   jax.ShapeDtypeStruct((48, 73), f32),
                   jax.ShapeDtypeStruct((256, 192), f32)),
    )(typeEmbed, tableEmbed, columnEmbed, opEmbed, joinEmbed, Wf, Wh, Wp)

    Ws_shift = jnp.concatenate(
        [jnp.zeros((37, _ES), f32), Ws, jnp.zeros((3, _ES), f32)], axis=0)
    weights = [T64, W48, Wf2, bf.reshape(1, -1), bf2.reshape(1, -1),
               Whh, bh.reshape(1, -1), Ws_shift, bs.reshape(1, -1),
               Wp[64:137, :], Wp[201:265, :], Wp[265:329, :],
               bp.reshape(1, -1)]

    grid = B // _BT
    return pl.pallas_call(
        _block,
        grid=(grid,),
        in_specs=[pl.BlockSpec((_BT, feature.shape[1]), lambda i: (i, 0))]
                 + [_full(w) for w in weights],
        out_specs=pl.BlockSpec((_BT, _ES), lambda i: (i, 0)),
        out_shape=jax.ShapeDtypeStruct((B, _ES), f32),
        compiler_params=pltpu.CompilerParams(
            dimension_semantics=("parallel",),
            vmem_limit_bytes=100 * 1024 * 1024),
    )(feature, *weights)


# speedup vs baseline: 1.0084x; 1.0084x over previous
import jax
import jax.numpy as jnp
from jax.experimental import pallas as pl
from jax.experimental.pallas import tpu as pltpu

_BT = 1024
_ES = 64


def _leaky(x):
    return jnp.maximum(x, 0.01 * x)


def _dot(a, b):
    return jax.lax.dot(a, b, preferred_element_type=jnp.float32)


def _prep(typeE_ref, tableE_ref, colE_ref, opE_ref, joinE_ref,
          Wf_ref, Wh_ref, Wp_ref, T64_ref, W256_ref, E_ref, Whh_ref):
    f32 = jnp.float32
    Wp = Wp_ref[...]
    Tt = _dot(typeE_ref[...], Wp[0:64, :])
    Tj = _dot(joinE_ref[...], Wp[137:201, :])
    Ttb = _dot(tableE_ref[...], Wp[201:265, :])
    T64_ref[...] = jnp.concatenate(
        [Tt, Tj, Ttb, jnp.zeros((14, 64), f32)], axis=0)

    Wf = Wf_ref[...]
    ct = _dot(colE_ref[...], Wf[0:64, :])
    ot = _dot(opE_ref[...], Wf[64:72, :])
    W256_ref[...] = jnp.concatenate(
        [ct, jnp.zeros((34, 73), f32), ot, jnp.zeros((54, 73), f32),
         Wf[72:73, :], jnp.zeros((127, 73), f32)], axis=0)

    r16 = jax.lax.broadcasted_iota(jnp.int32, (16, 1), 0)
    l768 = jax.lax.broadcasted_iota(jnp.int32, (1, 768), 1)
    seg = l768 // 256
    s = l768 % 256
    srcrow = 2 + seg + 3 * jnp.minimum(s // 64, 2)
    E_ref[...] = ((r16 == srcrow) & (s <= 128)).astype(f32)

    Wh = Wh_ref[...]
    r150 = jax.lax.broadcasted_iota(jnp.int32, (150, 1), 0)
    c50 = jax.lax.broadcasted_iota(jnp.int32, (1, 50), 1)
    rep = _dot((r150 // 3 == c50).astype(f32), Wh)
    mod3 = r150 % 3
    strided = jnp.concatenate([jnp.where(mod3 == 0, rep, 0.0),
                               jnp.where(mod3 == 1, rep, 0.0),
                               jnp.where(mod3 == 2, rep, 0.0)], axis=1)
    Whh_ref[...] = jnp.concatenate(
        [jnp.zeros((14, 192), f32), strided, jnp.zeros((92, 192), f32)], axis=0)


def _block(x_ref, T64_ref, W256_ref, E_ref, Wf2_ref, bf_ref, bf2_ref,
           Whh_ref, bh_ref, Ws_ref, bs_ref, Wpf_ref, Wptb_ref, Wph_ref,
           bp_ref, out_ref):
    f32 = jnp.float32
    i32 = jnp.int32

    ids = x_ref[:, 0:16]
    l64 = jax.lax.broadcasted_iota(i32, (1, 64), 1).astype(f32)
    oh64 = ((l64 == ids[:, 0:1]).astype(f32)
            + (l64 == ids[:, 1:2] + 20.0).astype(f32)
            + (l64 == x_ref[:, 164:165] + 30.0).astype(f32))
    acc = _dot(oh64, T64_ref[...])

    idsb = _dot(ids, E_ref[...])
    l768 = jax.lax.broadcasted_iota(i32, (1, 768), 1)
    s = l768 % 256
    P = jnp.where(s < 64, s, s - 64).astype(f32)
    cmask = (s < 128).astype(f32)
    vmask = (s == 128).astype(f32)
    oh_all = (idsb == P).astype(f32) * cmask + idsb * vmask
    m = [ids[:, 11 + j:12 + j] for j in range(3)]
    oh3 = jnp.concatenate([oh_all[:, 0:256], oh_all[:, 256:512],
                           oh_all[:, 512:768]], axis=0)
    Wf2 = Wf2_ref[...]
    bf = bf_ref[...]
    bf2 = bf2_ref[...]
    h1 = _leaky(_dot(oh3, W256_ref[...]) + bf)
    h2 = _leaky(_dot(h1, Wf2) + bf2)
    h2m = jnp.concatenate(m, axis=0) * h2
    facc = h2m[0:_BT] + h2m[_BT:2 * _BT] + h2m[2 * _BT:3 * _BT]
    msum = m[0] + m[1] + m[2]
    rnum = 1.0 / jnp.maximum(msum, 1.0)

    hist3 = _dot(x_ref[:, 0:256], Whh_ref[...])
    histEmb = (m[0] * hist3[:, 0:64] + m[1] * hist3[:, 64:128]
               + m[2] * hist3[:, 128:192] + msum * bh_ref[...]) * rnum

    samp = _dot(x_ref[:, 128:1165], Ws_ref[...][0:1037, :])

    pre = (acc
           + _dot(facc * rnum, Wpf_ref[...])
           + _dot(samp + bs_ref[...], Wptb_ref[...])
           + _dot(histEmb, Wph_ref[...])
           + bp_ref[...])
    out_ref[...] = _leaky(pre)


def _full(w):
    return pl.BlockSpec(w.shape, lambda i: tuple(0 for _ in w.shape))


def kernel(feature, typeEmbed, tableEmbed, columnEmbed, opEmbed, joinEmbed,
           Wf, bf, Wf2, bf2, Ws, bs, Wh, bh, Wp, bp):
    B = feature.shape[0]
    f32 = jnp.float32

    T64, W256, E, Whh = pl.pallas_call(
        _prep,
        out_shape=(jax.ShapeDtypeStruct((64, 64), f32),
                   jax.ShapeDtypeStruct((256, 73), f32),
                   jax.ShapeDtypeStruct((16, 768), f32),
                   jax.ShapeDtypeStruct((256, 192), f32)),
    )(typeEmbed, tableEmbed, columnEmbed, opEmbed, joinEmbed, Wf, Wh, Wp)

    Ws_shift = jnp.concatenate(
        [jnp.zeros((37, _ES), f32), Ws, jnp.zeros((3, _ES), f32)], axis=0)
    weights = [T64, W256, E, Wf2, bf.reshape(1, -1), bf2.reshape(1, -1),
               Whh, bh.reshape(1, -1), Ws_shift, bs.reshape(1, -1),
               Wp[64:137, :], Wp[201:265, :], Wp[265:329, :],
               bp.reshape(1, -1)]

    grid = B // _BT
    return pl.pallas_call(
        _block,
        grid=(grid,),
        in_specs=[pl.BlockSpec((_BT, feature.shape[1]), lambda i: (i, 0))]
                 + [_full(w) for w in weights],
        out_specs=pl.BlockSpec((_BT, _ES), lambda i: (i, 0)),
        out_shape=jax.ShapeDtypeStruct((B, _ES), f32),
        compiler_params=pltpu.CompilerParams(
            dimension_semantics=("parallel",),
            vmem_limit_bytes=100 * 1024 * 1024),
    )(feature, *weights)
